# baseline (speedup 1.0000x reference)
import jax
import jax.numpy as jnp
from jax import lax
from jax.experimental import pallas as pl
from jax.experimental.pallas import tpu as pltpu
from jax.experimental.pallas import tpu_sc as plsc

_NC = 2
_EI_R = 2 * 6400000 // 128
_EW_R = 6400000 // 128
_EI_PC = _EI_R // _NC
_EW_PC = _EW_R // _NC
_STEPS = 10
_EI_CH = _EI_PC // _STEPS
_EW_SIZES = [2496] * 9 + [25000 - 9 * 2496]
_EW_OFFS = [2496 * t for t in range(_STEPS)]
_EW_MAX = max(_EW_SIZES)


def _sc_copy_body(ei_in, ew_in, ei_out, ew_out,
                  ei_buf0, ei_buf1, ew_buf0, ew_buf1,
                  sei_in, sei_out, sew_in, sew_out):
    cid = lax.axis_index("c")
    ei_base = cid * _EI_PC
    ew_base = cid * _EW_PC
    ei_bufs = (ei_buf0, ei_buf1)
    ew_bufs = (ew_buf0, ew_buf1)

    def copies(t):
        b = t % 2
        ei_sl = pl.ds(ei_base + t * _EI_CH, _EI_CH)
        ew_sl = pl.ds(ew_base + _EW_OFFS[t], _EW_SIZES[t])
        ew_bsl = pl.ds(0, _EW_SIZES[t])
        return (
            pltpu.make_async_copy(ei_in.at[ei_sl], ei_bufs[b], sei_in.at[b]),
            pltpu.make_async_copy(ew_in.at[ew_sl], ew_bufs[b].at[ew_bsl],
                                  sew_in.at[b]),
            pltpu.make_async_copy(ei_bufs[b], ei_out.at[ei_sl], sei_out.at[b]),
            pltpu.make_async_copy(ew_bufs[b].at[ew_bsl], ew_out.at[ew_sl],
                                  sew_out.at[b]),
        )

    cei_in0, cew_in0, _, _ = copies(0)
    cei_in0.start()
    cew_in0.start()
    for t in range(_STEPS):
        cei_in, cew_in, cei_out, cew_out = copies(t)
        if t >= 1:
            _, _, pei_out, pew_out = copies(t - 1)
            pei_out.wait()
            pew_out.wait()
        if t + 1 < _STEPS:
            nei_in, new_in, _, _ = copies(t + 1)
            nei_in.start()
            new_in.start()
        cei_in.wait()
        cew_in.wait()
        cei_out.start()
        cew_out.start()
    for t in range(_STEPS - 1, _STEPS):
        _, _, cei_out, cew_out = copies(t)
        cei_out.wait()
        cew_out.wait()


_sc_copy = pl.kernel(
    _sc_copy_body,
    out_type=(
        jax.ShapeDtypeStruct((_EI_R, 128), jnp.int32),
        jax.ShapeDtypeStruct((_EW_R, 128), jnp.float32),
    ),
    mesh=plsc.ScalarSubcoreMesh(axis_name="c", num_cores=_NC),
    scratch_types=(
        pltpu.VMEM_SHARED((_EI_CH, 128), jnp.int32),
        pltpu.VMEM_SHARED((_EI_CH, 128), jnp.int32),
        pltpu.VMEM_SHARED((_EW_MAX, 128), jnp.float32),
        pltpu.VMEM_SHARED((_EW_MAX, 128), jnp.float32),
        pltpu.SemaphoreType.DMA((2,)),
        pltpu.SemaphoreType.DMA((2,)),
        pltpu.SemaphoreType.DMA((2,)),
        pltpu.SemaphoreType.DMA((2,)),
    ),
)


def kernel(ei, ew):
    ei_flat, ew_flat = _sc_copy(ei.reshape(_EI_R, 128), ew.reshape(_EW_R, 128))
    return ei_flat.reshape(ei.shape), ew_flat.reshape(ew.shape)


# speedup vs baseline: 1.1193x; 1.1193x over previous
import jax
import jax.numpy as jnp
from jax.experimental import pallas as pl
from jax.experimental.pallas import tpu as pltpu

_K = 25
_NBUF = 6
_DELAY = 2


def _copy_body(ei_ref, ew_ref, ei_out, ew_out,
               ei_buf, ew_buf, sei_in, sei_out, sew_in, sew_out):
    re_ = ei_ref.shape[0] // _K
    rw = ew_ref.shape[0] // _K

    def in_copies(k):
        s = k % _NBUF
        return (
            pltpu.make_async_copy(
                ei_ref.at[pl.ds(k * re_, re_), :], ei_buf.at[s], sei_in.at[s]),
            pltpu.make_async_copy(
                ew_ref.at[pl.ds(k * rw, rw), :], ew_buf.at[s], sew_in.at[s]),
        )

    def out_copies(k):
        s = k % _NBUF
        return (
            pltpu.make_async_copy(
                ei_buf.at[s], ei_out.at[pl.ds(k * re_, re_), :], sei_out.at[s]),
            pltpu.make_async_copy(
                ew_buf.at[s], ew_out.at[pl.ds(k * rw, rw), :], sew_out.at[s]),
        )

    for k in range(min(_NBUF, _K)):
        for c in in_copies(k):
            c.start(priority=k % 2)
    for k in range(_K):
        for c in in_copies(k):
            c.wait()
        for c in out_copies(k):
            c.start(priority=k % 2)
        j = k - _DELAY
        if j >= 0 and j + _NBUF < _K:
            for c in out_copies(j):
                c.wait()
            for c in in_copies(j + _NBUF):
                c.start(priority=(j + _NBUF) % 2)
    for j in range(max(0, _K - _NBUF), _K):
        for c in out_copies(j):
            c.wait()


def kernel(ei, ew):
    ei2 = ei.reshape(ei.size // 128, 128)
    ew2 = ew.reshape(ew.size // 128, 128)
    re_ = ei2.shape[0] // _K
    rw = ew2.shape[0] // _K
    out = pl.pallas_call(
        _copy_body,
        in_specs=(
            pl.BlockSpec(memory_space=pl.ANY),
            pl.BlockSpec(memory_space=pl.ANY),
        ),
        out_specs=(
            pl.BlockSpec(memory_space=pl.ANY),
            pl.BlockSpec(memory_space=pl.ANY),
        ),
        out_shape=(
            jax.ShapeDtypeStruct(ei2.shape, ei2.dtype),
            jax.ShapeDtypeStruct(ew2.shape, ew2.dtype),
        ),
        scratch_shapes=(
            pltpu.VMEM((_NBUF, re_, 128), ei.dtype),
            pltpu.VMEM((_NBUF, rw, 128), ew.dtype),
            pltpu.SemaphoreType.DMA((_NBUF,)),
            pltpu.SemaphoreType.DMA((_NBUF,)),
            pltpu.SemaphoreType.DMA((_NBUF,)),
            pltpu.SemaphoreType.DMA((_NBUF,)),
        ),
    )(ei2, ew2)
    return out[0].reshape(ei.shape), out[1].reshape(ew.shape)
